# baseline (speedup 1.0000x reference)
import functools

import jax
import jax.numpy as jnp
from jax import lax
from jax.experimental import pallas as pl
from jax.experimental.pallas import tpu as pltpu
from jax.experimental.pallas import tpu_sc as plsc

NC, NS = 2, 16
NW = NC * NS
ROW = 128
K = 4
HID = 64


@functools.lru_cache(maxsize=None)
def _make_detile(vocab: int, hid: int):
    nblk = vocab // 128
    rem = vocab - nblk * 128
    mesh = plsc.VectorSubcoreMesh(core_axis_name="c", subcore_axis_name="s")

    @functools.partial(
        pl.kernel,
        out_type=jax.ShapeDtypeStruct((vocab // 2, 128), jnp.float32),
        mesh=mesh,
        scratch_types=[
            pltpu.VMEM((2, hid, 128), jnp.float32),
            pltpu.VMEM((2, 64, 128), jnp.float32),
            pltpu.SemaphoreType.DMA,
            pltpu.SemaphoreType.DMA,
            pltpu.SemaphoreType.DMA,
            pltpu.SemaphoreType.DMA,
        ],
        compiler_params=pltpu.CompilerParams(needs_layout_passes=False),
    )
    def k(wt_hbm, rem_hbm, out_hbm, stage_v, tr_v, gs0, gs1, ws0, ws1):
        iota = lax.iota(jnp.int32, 16)
        wid = lax.axis_index("s") * NC + lax.axis_index("c")
        lo = wid * nblk // NW
        hi = (wid + 1) * nblk // NW
        gsem = (gs0, gs1)
        wsem = (ws0, ws1)

        def fire_stage(blk, b):
            pltpu.async_copy(
                wt_hbm.at[:, pl.ds(blk * 128, 128)], stage_v.at[b], gsem[b])

        def transpose(b):
            @pl.loop(0, 128)
            def _v(v):
                vvec = jnp.full((16,), 0, jnp.int32) + v
                for q in range(hid // 16):
                    g = plsc.load_gather(
                        stage_v.at[b], [q * 16 + iota, vvec])
                    tr_v[b, v >> 1, pl.ds((v & 1) * 64 + q * 16, 16)] = g

        for b in range(2):
            @pl.when(lo + b < hi)
            def _():
                fire_stage(lo + b, b)

        @pl.loop(0, (hi - lo + 1) // 2 * 2, step=2)
        def _blk(i):
            for b in range(2):
                blk = lo + i + b

                @pl.when(blk < hi)
                def _():
                    pltpu.make_async_copy(
                        wt_hbm.at[:, pl.ds(blk * 128, 128)],
                        stage_v.at[b], gsem[b]).wait()
                    @pl.when(blk >= lo + 2)
                    def _():
                        pltpu.make_async_copy(
                            tr_v.at[b], out_hbm.at[pl.ds(0, 64)],
                            wsem[b]).wait()
                    transpose(b)
                    @pl.when(blk + 2 < hi)
                    def _():
                        fire_stage(blk + 2, b)
                    pltpu.async_copy(
                        tr_v.at[b], out_hbm.at[pl.ds(blk * 64, 64)],
                        wsem[b])
        for b in range(2):
            @pl.when(lo + b < hi)
            def _():
                pltpu.make_async_copy(
                    tr_v.at[b], out_hbm.at[pl.ds(0, 64)],
                    wsem[b]).wait()

        if rem:
            @pl.when(wid == NW - 1)
            def _():
                pltpu.sync_copy(rem_hbm, tr_v.at[0, pl.ds(0, rem // 2)])
                pltpu.sync_copy(
                    tr_v.at[0, pl.ds(0, rem // 2)],
                    out_hbm.at[pl.ds(nblk * 64, rem // 2)])

    return k


@functools.lru_cache(maxsize=None)
def _make_gather(n_rows: int):
    rows_per_w = n_rows // NW
    groups = rows_per_w // K
    mesh = plsc.VectorSubcoreMesh(core_axis_name="c", subcore_axis_name="s")

    @functools.partial(
        pl.kernel,
        out_type=jax.ShapeDtypeStruct((n_rows, ROW, HID), jnp.float32),
        mesh=mesh,
        scratch_types=[
            pltpu.VMEM((K, ROW), jnp.int32),
            pltpu.VMEM((K, ROW, HID), jnp.float32),
            pltpu.SemaphoreType.DMA,
        ],
        compiler_params=pltpu.CompilerParams(use_tc_tiling_on_sc=False),
    )
    def k(ids_hbm, table_hbm, out_hbm, idx_v, rows_v, sem):
        wid = lax.axis_index("s") * NC + lax.axis_index("c")
        row_base = wid * rows_per_w

        @pl.loop(0, groups)
        def _group(g):
            r0 = row_base + g * K
            pltpu.sync_copy(ids_hbm.at[pl.ds(r0, K)], idx_v)
            cps = [
                pltpu.async_copy(table_hbm.at[idx_v.at[j]], rows_v.at[j], sem)
                for j in range(K)
            ]
            for cp in cps:
                cp.wait()
            pltpu.sync_copy(rows_v, out_hbm.at[pl.ds(r0, K)])

    return k


def kernel(input_ids, wte):
    B, L = input_ids.shape
    V, H = wte.shape
    n = B * L
    n_rows = n // ROW
    ids = input_ids.reshape(n_rows, ROW).astype(jnp.int32)
    nblk = V // 128
    rem = V - nblk * 128
    wrem = wte[V - rem:].reshape(max(rem // 2, 1), 128) if rem else wte[:1]
    table = _make_detile(V, H)(wte.T, wrem).reshape(V, H)
    out = _make_gather(n_rows)(ids, table)
    return out.reshape(B, L, HID)


# speedup vs baseline: 2.0249x; 2.0249x over previous
import functools

import jax
import jax.numpy as jnp
from jax import lax
from jax.experimental import pallas as pl
from jax.experimental.pallas import tpu as pltpu
from jax.experimental.pallas import tpu_sc as plsc

NC, NS = 2, 16
NW = NC * NS
ROW = 128
K = 8
HID = 64


@functools.lru_cache(maxsize=None)
def _make_kernel(n_rows: int):
    rows_per_w = n_rows // NW
    groups = rows_per_w // K
    mesh = plsc.VectorSubcoreMesh(core_axis_name="c", subcore_axis_name="s")

    @functools.partial(
        pl.kernel,
        out_type=jax.ShapeDtypeStruct((n_rows, ROW, HID), jnp.float32),
        mesh=mesh,
        scratch_types=[
            pltpu.VMEM((K, ROW), jnp.int32),
            pltpu.VMEM((K, ROW, HID), jnp.float32),
            pltpu.SemaphoreType.DMA,
        ],
        compiler_params=pltpu.CompilerParams(use_tc_tiling_on_sc=False),
    )
    def k(ids_hbm, table_hbm, out_hbm, idx_v, rows_v, sem):
        wid = lax.axis_index("s") * NC + lax.axis_index("c")
        row_base = wid * rows_per_w

        @pl.loop(0, groups)
        def _group(g):
            r0 = row_base + g * K
            pltpu.sync_copy(ids_hbm.at[pl.ds(r0, K)], idx_v)
            cps = [
                pltpu.async_copy(table_hbm.at[idx_v.at[j]], rows_v.at[j], sem)
                for j in range(K)
            ]
            for cp in cps:
                cp.wait()
            pltpu.sync_copy(rows_v, out_hbm.at[pl.ds(r0, K)])

    return k


def kernel(input_ids, wte):
    B, L = input_ids.shape
    n = B * L
    n_rows = n // ROW
    ids = input_ids.reshape(n_rows, ROW).astype(jnp.int32)
    out = _make_kernel(n_rows)(ids, wte)
    return out.reshape(B, L, HID)
